# Initial kernel scaffold; baseline (speedup 1.0000x reference)
import jax
import jax.numpy as jnp
from jax.experimental import pallas as pl


def kernel(rois, pts, pts_feature):
    raise NotImplementedError("write your pallas kernel here")


# speedup vs baseline: 10.4659x; 10.4659x over previous
import functools

import jax
import jax.numpy as jnp
from jax import lax
from jax.experimental import pallas as pl
from jax.experimental.pallas import tpu as pltpu
from jax.experimental.pallas import tpu_sc as plsc

OUT_SIZE = 12
V = OUT_SIZE * OUT_SIZE * OUT_SIZE
NC = 2
NS = 16
NW = NC * NS
L = 16
BLK = 2048

NEG_INF = float("-inf")


def _pool_body(nrois, npoints, C, n_rounds,
               prm_hbm, xs_hbm, ys_hbm, zs_hbm, pf_hbm, out_hbm,
               acc, xb, yb, zb, rows, prm, idxs, gsem):
  acc_words = V * C
  n_vec = acc_words // L
  wid = lax.axis_index("s") * NC + lax.axis_index("c")

  for t in range(n_rounds):
    r = wid + t * NW

    @pl.when(r < nrois)
    def _do_roi():
      pltpu.sync_copy(prm_hbm.at[r], prm)
      pv = prm[...]
      cx = pv[0]
      cy = pv[1]
      czc = pv[2]
      hdx = pv[3]
      hdy = pv[4]
      hdz = pv[5]
      cosa = pv[6]
      sina = pv[7]
      xres = pv[8]
      yres = pv[9]
      zres = pv[10]
      cz = pv[11]
      r2 = pv[12]

      def init_body(i, _):
        acc[pl.ds(i * L, L)] = jnp.full((L,), NEG_INF, jnp.float32)
        return _
      lax.fori_loop(0, n_vec, init_body, None)

      def blk_body(b, _):
        base = b * BLK
        pltpu.sync_copy(xs_hbm.at[pl.ds(base, BLK)], xb)
        pltpu.sync_copy(ys_hbm.at[pl.ds(base, BLK)], yb)
        pltpu.sync_copy(zs_hbm.at[pl.ds(base, BLK)], zb)

        def chunk_body(ci, _):
          off = ci * L
          x = xb[pl.ds(off, L)]
          y = yb[pl.ds(off, L)]
          z = zb[pl.ds(off, L)]
          sx = x - cx
          sy = y - cy
          zz = z - czc
          zok = jnp.abs(zz) <= hdz
          pre = ((sx * sx + sy * sy) <= r2) & zok

          npre = plsc.all_reduce_population_count(pre)

          @pl.when(npre[0] > 0)
          def _full():
            lx = sx * cosa - sy * sina
            ly = sx * sina + sy * cosa
            inb = zok & (jnp.abs(lx) < hdx) & (jnp.abs(ly) < hdy)

            ninb = plsc.all_reduce_population_count(inb)

            @pl.when(ninb[0] > 0)
            def _scatter():
              fx = (lx + hdx) / xres
              fy = (ly + hdy) / yres
              fz = (z - cz) / zres
              xi = jnp.clip(fx, 0.0, float(OUT_SIZE - 1)).astype(jnp.int32)
              yi = jnp.clip(fy, 0.0, float(OUT_SIZE - 1)).astype(jnp.int32)
              zi = jnp.clip(fz, 0.0, float(OUT_SIZE - 1)).astype(jnp.int32)
              seg = jnp.where(inb, (xi * OUT_SIZE + yi) * OUT_SIZE + zi, 0)
              msk = jnp.where(inb, 1, 0)
              pidx = base + off + lax.iota(jnp.int32, L)
              idxs[...] = jnp.where(inb, pidx, 0)
              pltpu.async_copy(pf_hbm.at[idxs], rows, gsem).wait()
              for j in range(L):
                @pl.when(msk[j] > 0)
                def _upd(j=j):
                  rb = seg[j] * C
                  for cb in range(C // L):
                    sl = pl.ds(rb + cb * L, L)
                    acc[sl] = jnp.maximum(acc[sl], rows[j, pl.ds(cb * L, L)])
          return _
        lax.fori_loop(0, BLK // L, chunk_body, None)
        return _
      lax.fori_loop(0, npoints // BLK, blk_body, None)

      def fin_body(i, _):
        sl = pl.ds(i * L, L)
        v = acc[sl]
        acc[sl] = jnp.where(v == NEG_INF, 0.0, v)
        return _
      lax.fori_loop(0, n_vec, fin_body, None)

      pltpu.sync_copy(acc, out_hbm.at[r])


def kernel(rois, pts, pts_feature):
  nrois = rois.shape[0]
  npoints = pts.shape[0]
  C = pts_feature.shape[1]
  assert npoints % BLK == 0 and C % L == 0
  n_rounds = -(-nrois // NW)

  cx, cy, cz = rois[:, 0], rois[:, 1], rois[:, 2]
  dx, dy, dz = rois[:, 3], rois[:, 4], rois[:, 5]
  rz = rois[:, 6]
  czc = cz + dz * 0.5
  cosa = jnp.cos(-rz)
  sina = jnp.sin(-rz)
  hdx, hdy, hdz = dx * 0.5, dy * 0.5, dz * 0.5
  xres = dx / OUT_SIZE
  yres = dy / OUT_SIZE
  zres = dz / OUT_SIZE
  r2 = hdx * hdx + hdy * hdy
  pad = jnp.zeros((nrois,), jnp.float32)
  prm = jnp.stack(
      [cx, cy, czc, hdx, hdy, hdz, cosa, sina, xres, yres, zres, cz, r2,
       pad, pad, pad], axis=1)

  xs = jnp.asarray(pts[:, 0], jnp.float32)
  ys = jnp.asarray(pts[:, 1], jnp.float32)
  zs = jnp.asarray(pts[:, 2], jnp.float32)

  mesh = plsc.VectorSubcoreMesh(
      core_axis_name="c", subcore_axis_name="s",
      num_cores=NC, num_subcores=NS)

  fn = pl.kernel(
      functools.partial(_pool_body, nrois, npoints, C, n_rounds),
      out_type=jax.ShapeDtypeStruct((nrois, V * C), jnp.float32),
      mesh=mesh,
      compiler_params=pltpu.CompilerParams(
          needs_layout_passes=False, use_tc_tiling_on_sc=False),
      scratch_types=[
          pltpu.VMEM((V * C,), jnp.float32),
          pltpu.VMEM((BLK,), jnp.float32),
          pltpu.VMEM((BLK,), jnp.float32),
          pltpu.VMEM((BLK,), jnp.float32),
          pltpu.VMEM((L, C), jnp.float32),
          pltpu.VMEM((L,), jnp.float32),
          pltpu.VMEM((L,), jnp.int32),
          pltpu.SemaphoreType.DMA,
      ],
  )
  out = fn(prm, xs, ys, zs, pts_feature)
  return out.reshape(nrois, OUT_SIZE, OUT_SIZE, OUT_SIZE, C)


# speedup vs baseline: 11.4235x; 1.0915x over previous
import functools

import jax
import jax.numpy as jnp
from jax import lax
from jax.experimental import pallas as pl
from jax.experimental.pallas import tpu as pltpu
from jax.experimental.pallas import tpu_sc as plsc

OUT_SIZE = 12
V = OUT_SIZE * OUT_SIZE * OUT_SIZE
NC = 2
NS = 16
NW = NC * NS
L = 16
BLK = 4096
SUB = 4
UNR = 8

NEG_INF = float("-inf")


def _pool_body(nrois, npoints, C, n_rounds,
               prm_hbm, xs_hbm, ys_hbm, zs_hbm, pf_hbm, out_hbm,
               acc, xb, yb, zb, rows, prm, idxs, gsem):
  acc_words = V * C
  n_vec = acc_words // L
  wid = lax.axis_index("s") * NC + lax.axis_index("c")

  for t in range(n_rounds):
    r = wid + t * NW

    @pl.when(r < nrois)
    def _do_roi():
      pltpu.sync_copy(prm_hbm.at[r], prm)
      pv = prm[...]
      cx = pv[0]
      cy = pv[1]
      czc = pv[2]
      hdx = pv[3]
      hdy = pv[4]
      hdz = pv[5]
      cosa = pv[6]
      sina = pv[7]
      xres = pv[8]
      yres = pv[9]
      zres = pv[10]
      cz = pv[11]
      r2 = pv[12]

      neg = jnp.full((L,), NEG_INF, jnp.float32)

      def init_body(i, _):
        for u in range(UNR):
          acc[pl.ds((i * UNR + u) * L, L)] = neg
        return _
      lax.fori_loop(0, n_vec // UNR, init_body, None)

      def blk_body(b, _):
        base = b * BLK
        pltpu.sync_copy(xs_hbm.at[pl.ds(base, BLK)], xb)
        pltpu.sync_copy(ys_hbm.at[pl.ds(base, BLK)], yb)
        pltpu.sync_copy(zs_hbm.at[pl.ds(base, BLK)], zb)

        def chunk_body(ci, _):
          off0 = ci * (L * SUB)
          zs_ = []
          pres = []
          sxs = []
          sys_ = []
          for k in range(SUB):
            off = off0 + k * L
            x = xb[pl.ds(off, L)]
            y = yb[pl.ds(off, L)]
            z = zb[pl.ds(off, L)]
            sx = x - cx
            sy = y - cy
            zok = jnp.abs(z - czc) <= hdz
            pre = ((sx * sx + sy * sy) <= r2) & zok
            zs_.append(z)
            sxs.append(sx)
            sys_.append(sy)
            pres.append(pre)
          anypre = pres[0]
          for k in range(1, SUB):
            anypre = anypre | pres[k]
          npre = plsc.all_reduce_population_count(anypre)

          @pl.when(npre[0] > 0)
          def _full():
            for k in range(SUB):
              off = off0 + k * L
              sx, sy, z = sxs[k], sys_[k], zs_[k]

              @pl.when(plsc.all_reduce_population_count(pres[k])[0] > 0)
              def _one(sx=sx, sy=sy, z=z, pre=pres[k], off=off):
                lx = sx * cosa - sy * sina
                ly = sx * sina + sy * cosa
                inb = pre & (jnp.abs(lx) < hdx) & (jnp.abs(ly) < hdy)

                ninb = plsc.all_reduce_population_count(inb)

                @pl.when(ninb[0] > 0)
                def _scatter():
                  fx = (lx + hdx) / xres
                  fy = (ly + hdy) / yres
                  fz = (z - cz) / zres
                  xi = jnp.clip(fx, 0.0, float(OUT_SIZE - 1)).astype(jnp.int32)
                  yi = jnp.clip(fy, 0.0, float(OUT_SIZE - 1)).astype(jnp.int32)
                  zi = jnp.clip(fz, 0.0, float(OUT_SIZE - 1)).astype(jnp.int32)
                  seg = jnp.where(inb, (xi * OUT_SIZE + yi) * OUT_SIZE + zi, 0)
                  msk = jnp.where(inb, 1, 0)
                  pidx = base + off + lax.iota(jnp.int32, L)
                  idxs[...] = jnp.where(inb, pidx, 0)
                  pltpu.async_copy(pf_hbm.at[idxs], rows, gsem).wait()
                  for j in range(L):
                    @pl.when(msk[j] > 0)
                    def _upd(j=j):
                      rb = seg[j] * C
                      for cb in range(C // L):
                        sl = pl.ds(rb + cb * L, L)
                        acc[sl] = jnp.maximum(acc[sl],
                                              rows[j, pl.ds(cb * L, L)])
          return _
        lax.fori_loop(0, BLK // (L * SUB), chunk_body, None)
        return _
      lax.fori_loop(0, npoints // BLK, blk_body, None)

      def fin_body(i, _):
        for u in range(UNR):
          sl = pl.ds((i * UNR + u) * L, L)
          v = acc[sl]
          acc[sl] = jnp.where(v == NEG_INF, 0.0, v)
        return _
      lax.fori_loop(0, n_vec // UNR, fin_body, None)

      pltpu.sync_copy(acc, out_hbm.at[r])


def kernel(rois, pts, pts_feature):
  nrois = rois.shape[0]
  npoints = pts.shape[0]
  C = pts_feature.shape[1]
  assert npoints % BLK == 0 and C % L == 0
  n_rounds = -(-nrois // NW)

  cx, cy, cz = rois[:, 0], rois[:, 1], rois[:, 2]
  dx, dy, dz = rois[:, 3], rois[:, 4], rois[:, 5]
  rz = rois[:, 6]
  czc = cz + dz * 0.5
  cosa = jnp.cos(-rz)
  sina = jnp.sin(-rz)
  hdx, hdy, hdz = dx * 0.5, dy * 0.5, dz * 0.5
  xres = dx / OUT_SIZE
  yres = dy / OUT_SIZE
  zres = dz / OUT_SIZE
  r2 = hdx * hdx + hdy * hdy
  pad = jnp.zeros((nrois,), jnp.float32)
  prm = jnp.stack(
      [cx, cy, czc, hdx, hdy, hdz, cosa, sina, xres, yres, zres, cz, r2,
       pad, pad, pad], axis=1)

  xs = jnp.asarray(pts[:, 0], jnp.float32)
  ys = jnp.asarray(pts[:, 1], jnp.float32)
  zs = jnp.asarray(pts[:, 2], jnp.float32)

  mesh = plsc.VectorSubcoreMesh(
      core_axis_name="c", subcore_axis_name="s",
      num_cores=NC, num_subcores=NS)

  fn = pl.kernel(
      functools.partial(_pool_body, nrois, npoints, C, n_rounds),
      out_type=jax.ShapeDtypeStruct((nrois, V * C), jnp.float32),
      mesh=mesh,
      compiler_params=pltpu.CompilerParams(
          needs_layout_passes=False, use_tc_tiling_on_sc=False),
      scratch_types=[
          pltpu.VMEM((V * C,), jnp.float32),
          pltpu.VMEM((BLK,), jnp.float32),
          pltpu.VMEM((BLK,), jnp.float32),
          pltpu.VMEM((BLK,), jnp.float32),
          pltpu.VMEM((L, C), jnp.float32),
          pltpu.VMEM((L,), jnp.float32),
          pltpu.VMEM((L,), jnp.int32),
          pltpu.SemaphoreType.DMA,
      ],
  )
  out = fn(prm, xs, ys, zs, pts_feature)
  return out.reshape(nrois, OUT_SIZE, OUT_SIZE, OUT_SIZE, C)


# speedup vs baseline: 28.9784x; 2.5367x over previous
import functools

import jax
import jax.numpy as jnp
from jax import lax
from jax.experimental import pallas as pl
from jax.experimental.pallas import tpu as pltpu
from jax.experimental.pallas import tpu_sc as plsc

OUT_SIZE = 12
V = OUT_SIZE * OUT_SIZE * OUT_SIZE
NC = 2
NS = 16
NW = NC * NS
L = 16
BLK = 2048
SUB = 4
UNR = 8
CAP = 96
FLUSH_AT = CAP - SUB * L

NEG_INF = float("-inf")


def _pool_body(nrois, npoints, C, n_rounds,
               prm_hbm, coords_hbm, pf_hbm, out_hbm,
               acc, cbuf, rowsf, prm, segl, pidxl, cnt_ref,
               sem0, sem1, gsem):
  acc_words = V * C
  n_vec = acc_words // L
  nblk = npoints // BLK
  assert nblk % 2 == 0
  wid = lax.axis_index("s") * NC + lax.axis_index("c")

  def blk_copy(b, buf, sem):
    return pltpu.make_async_copy(coords_hbm.at[b], cbuf.at[buf], sem)

  for t in range(n_rounds):
    r = wid + t * NW

    @pl.when(r < nrois)
    def _do_roi():
      pltpu.sync_copy(prm_hbm.at[r], prm)
      pv = prm[...]
      cx = pv[0]
      cy = pv[1]
      czc = pv[2]
      hdx = pv[3]
      hdy = pv[4]
      hdz = pv[5]
      cosa = pv[6]
      sina = pv[7]
      ixres = pv[8]
      iyres = pv[9]
      izres = pv[10]
      cz = pv[11]

      neg = jnp.full((L,), NEG_INF, jnp.float32)
      zero_i = jnp.zeros((L,), jnp.int32)

      for g in range(CAP // L):
        pidxl[pl.ds(g * L, L)] = zero_i
      cnt_ref[0] = 0

      def init_body(i, _):
        for u in range(UNR):
          acc[pl.ds((i * UNR + u) * L, L)] = neg
        return _
      lax.fori_loop(0, n_vec // UNR, init_body, None)

      def flush(n):
        pltpu.async_copy(pf_hbm.at[pidxl], rowsf, gsem).wait()
        ngr = (n + (L - 1)) // L

        def group_body(g, _):
          gb = g * L
          seg16 = segl[pl.ds(gb, L)]
          valid = jnp.where(lax.iota(jnp.int32, L) < (n - gb), 1, 0)
          for j in range(L):
            @pl.when(valid[j] > 0)
            def _upd(j=j):
              rb = seg16[j] * C
              for cb in range(C // L):
                sl = pl.ds(rb + cb * L, L)
                acc[sl] = jnp.maximum(acc[sl], rowsf[gb + j, pl.ds(cb * L, L)])
          return _
        lax.fori_loop(0, ngr, group_body, None)
        cnt_ref[0] = 0

      def scan_block(buf, base):
        iota3 = lax.iota(jnp.int32, L) * 3

        def chunk_body(ci, _):
          off0 = ci * (L * SUB)
          zs_ = []
          lxs = []
          lys = []
          inbs = []
          for k in range(SUB):
            off = off0 + k * L
            xidx = off * 3 + iota3
            x = plsc.load_gather(cbuf.at[buf], [xidx])
            y = plsc.load_gather(cbuf.at[buf], [xidx + 1])
            z = plsc.load_gather(cbuf.at[buf], [xidx + 2])
            sx = x - cx
            sy = y - cy
            zok = jnp.abs(z - czc) <= hdz
            lx = sx * cosa - sy * sina
            ly = sx * sina + sy * cosa
            inb = zok & (jnp.abs(lx) < hdx) & (jnp.abs(ly) < hdy)
            zs_.append(z)
            lxs.append(lx)
            lys.append(ly)
            inbs.append(inb)
          pk = plsc.all_reduce_population_count(inbs[0])
          for k in range(1, SUB):
            pk = pk | (plsc.all_reduce_population_count(inbs[k]) << (8 * k))
          n_all = pk[0]

          @pl.when(n_all != 0)
          def _collect_all():
            for k in range(SUB):
              nk = (n_all >> (8 * k)) & 0xFF

              @pl.when(nk > 0)
              def _one(lx=lxs[k], ly=lys[k], z=zs_[k], inb=inbs[k],
                       off=off0 + k * L, nk=nk):
                fx = (lx + hdx) * ixres
                fy = (ly + hdy) * iyres
                fz = (z - cz) * izres
                xi = jnp.clip(fx, 0.0, float(OUT_SIZE - 1)).astype(jnp.int32)
                yi = jnp.clip(fy, 0.0, float(OUT_SIZE - 1)).astype(jnp.int32)
                zi = jnp.clip(fz, 0.0, float(OUT_SIZE - 1)).astype(jnp.int32)
                seg = (xi * OUT_SIZE + yi) * OUT_SIZE + zi
                pidx = base + off + lax.iota(jnp.int32, L)
                n0 = cnt_ref[0]
                plsc.store_compressed(segl.at[pl.ds(n0, L)], seg, mask=inb)
                plsc.store_compressed(pidxl.at[pl.ds(n0, L)], pidx, mask=inb)
                cnt_ref[0] = n0 + nk

            @pl.when(cnt_ref[0] > FLUSH_AT)
            def _flush_now():
              flush(cnt_ref[0])
          return _
        lax.fori_loop(0, BLK // (L * SUB), chunk_body, None)

      blk_copy(0, 0, sem0).start()

      def pair_body(bb, _):
        b0 = 2 * bb
        blk_copy(b0 + 1, 1, sem1).start()
        blk_copy(b0, 0, sem0).wait()
        scan_block(0, b0 * BLK)

        @pl.when(b0 + 2 < nblk)
        def _prefetch():
          blk_copy(b0 + 2, 0, sem0).start()
        blk_copy(b0 + 1, 1, sem1).wait()
        scan_block(1, (b0 + 1) * BLK)
        return _
      lax.fori_loop(0, nblk // 2, pair_body, None)

      @pl.when(cnt_ref[0] > 0)
      def _final_flush():
        flush(cnt_ref[0])

      def fin_body(i, _):
        for u in range(UNR):
          sl = pl.ds((i * UNR + u) * L, L)
          v = acc[sl]
          acc[sl] = jnp.where(v == NEG_INF, 0.0, v)
        return _
      lax.fori_loop(0, n_vec // UNR, fin_body, None)

      pltpu.sync_copy(acc, out_hbm.at[r])


def kernel(rois, pts, pts_feature):
  nrois = rois.shape[0]
  npoints = pts.shape[0]
  C = pts_feature.shape[1]
  assert npoints % (2 * BLK) == 0 and C % L == 0
  n_rounds = -(-nrois // NW)
  nblk = npoints // BLK

  cx, cy, cz = rois[:, 0], rois[:, 1], rois[:, 2]
  dx, dy, dz = rois[:, 3], rois[:, 4], rois[:, 5]
  rz = rois[:, 6]
  czc = cz + dz * 0.5
  cosa = jnp.cos(-rz)
  sina = jnp.sin(-rz)
  hdx, hdy, hdz = dx * 0.5, dy * 0.5, dz * 0.5
  ixres = OUT_SIZE / dx
  iyres = OUT_SIZE / dy
  izres = OUT_SIZE / dz
  pad = jnp.zeros((nrois,), jnp.float32)
  prm = jnp.stack(
      [cx, cy, czc, hdx, hdy, hdz, cosa, sina, ixres, iyres, izres, cz,
       pad, pad, pad, pad], axis=1)

  coords = pts.reshape(nblk, 3 * BLK)

  mesh = plsc.VectorSubcoreMesh(
      core_axis_name="c", subcore_axis_name="s",
      num_cores=NC, num_subcores=NS)

  fn = pl.kernel(
      functools.partial(_pool_body, nrois, npoints, C, n_rounds),
      out_type=jax.ShapeDtypeStruct((nrois, V * C), jnp.float32),
      mesh=mesh,
      compiler_params=pltpu.CompilerParams(
          needs_layout_passes=False, use_tc_tiling_on_sc=False),
      scratch_types=[
          pltpu.VMEM((V * C,), jnp.float32),
          pltpu.VMEM((2, 3 * BLK), jnp.float32),
          pltpu.VMEM((CAP, C), jnp.float32),
          pltpu.VMEM((L,), jnp.float32),
          pltpu.VMEM((CAP,), jnp.int32),
          pltpu.VMEM((CAP,), jnp.int32),
          pltpu.SMEM((1,), jnp.int32),
          pltpu.SemaphoreType.DMA,
          pltpu.SemaphoreType.DMA,
          pltpu.SemaphoreType.DMA,
      ],
  )
  out = fn(prm, coords, pts_feature)
  return out.reshape(nrois, OUT_SIZE, OUT_SIZE, OUT_SIZE, C)


# speedup vs baseline: 31.8593x; 1.0994x over previous
import functools

import jax
import jax.numpy as jnp
from jax import lax
from jax.experimental import pallas as pl
from jax.experimental.pallas import tpu as pltpu
from jax.experimental.pallas import tpu_sc as plsc

OUT_SIZE = 12
V = OUT_SIZE * OUT_SIZE * OUT_SIZE
NC = 2
NS = 16
NW = NC * NS
L = 16
BLK = 2048
SUB = 4
UNR = 8
CAP = 96
FLUSH_AT = CAP - SUB * L

NEG_INF = float("-inf")


def _pool_body(nrois, npoints, C, n_rounds,
               prm_hbm, coords_hbm, pf_hbm, out_hbm,
               acc, cbuf, rowsf, prm, segl, pidxl, cnt_ref,
               sem0, sem1, gsem):
  acc_words = V * C
  n_vec = acc_words // L
  nblk = npoints // BLK
  assert nblk % 2 == 0
  wid = lax.axis_index("s") * NC + lax.axis_index("c")

  def blk_copy(b, buf, sem):
    return pltpu.make_async_copy(coords_hbm.at[b], cbuf.at[buf], sem)

  for t in range(n_rounds):
    r = wid + t * NW

    @pl.when(r < nrois)
    def _do_roi():
      pltpu.sync_copy(prm_hbm.at[r], prm)
      pv = prm[...]
      cx = pv[0]
      cy = pv[1]
      czc = pv[2]
      hdx = pv[3]
      hdy = pv[4]
      hdz = pv[5]
      cosa = pv[6]
      sina = pv[7]
      ixres = pv[8]
      iyres = pv[9]
      izres = pv[10]
      cz = pv[11]

      neg = jnp.full((L,), NEG_INF, jnp.float32)
      zero_i = jnp.zeros((L,), jnp.int32)

      for g in range(CAP // L):
        pidxl[pl.ds(g * L, L)] = zero_i
      cnt_ref[0] = 0

      def init_body(i, _):
        for u in range(UNR):
          acc[pl.ds((i * UNR + u) * L, L)] = neg
        return _
      lax.fori_loop(0, n_vec // UNR, init_body, None)

      def flush(n):
        pltpu.async_copy(pf_hbm.at[pidxl], rowsf, gsem).wait()
        ngr = (n + (L - 1)) // L

        def group_body(g, _):
          gb = g * L
          seg16 = segl[pl.ds(gb, L)]
          valid = jnp.where(lax.iota(jnp.int32, L) < (n - gb), 1, 0)
          for j in range(L):
            @pl.when(valid[j] > 0)
            def _upd(j=j):
              rb = seg16[j] * C
              for cb in range(C // L):
                sl = pl.ds(rb + cb * L, L)
                acc[sl] = jnp.maximum(acc[sl], rowsf[gb + j, pl.ds(cb * L, L)])
          return _
        lax.fori_loop(0, ngr, group_body, None)
        cnt_ref[0] = 0

      def scan_block(buf, base):
        def chunk_body(ci, _):
          off0 = ci * (L * SUB)
          zs_ = []
          lxs = []
          lys = []
          inbs = []
          for k in range(SUB):
            off = off0 + k * L
            x = cbuf[buf, 0, pl.ds(off, L)]
            y = cbuf[buf, 1, pl.ds(off, L)]
            z = cbuf[buf, 2, pl.ds(off, L)]
            sx = x - cx
            sy = y - cy
            zok = jnp.abs(z - czc) <= hdz
            lx = sx * cosa - sy * sina
            ly = sx * sina + sy * cosa
            inb = zok & (jnp.abs(lx) < hdx) & (jnp.abs(ly) < hdy)
            zs_.append(z)
            lxs.append(lx)
            lys.append(ly)
            inbs.append(inb)
          pk = plsc.all_reduce_population_count(inbs[0])
          for k in range(1, SUB):
            pk = pk | (plsc.all_reduce_population_count(inbs[k]) << (8 * k))
          n_all = pk[0]

          @pl.when(n_all != 0)
          def _collect_all():
            for k in range(SUB):
              nk = (n_all >> (8 * k)) & 0xFF

              @pl.when(nk > 0)
              def _one(lx=lxs[k], ly=lys[k], z=zs_[k], inb=inbs[k],
                       off=off0 + k * L, nk=nk):
                fx = (lx + hdx) * ixres
                fy = (ly + hdy) * iyres
                fz = (z - cz) * izres
                xi = jnp.clip(fx, 0.0, float(OUT_SIZE - 1)).astype(jnp.int32)
                yi = jnp.clip(fy, 0.0, float(OUT_SIZE - 1)).astype(jnp.int32)
                zi = jnp.clip(fz, 0.0, float(OUT_SIZE - 1)).astype(jnp.int32)
                seg = (xi * OUT_SIZE + yi) * OUT_SIZE + zi
                pidx = base + off + lax.iota(jnp.int32, L)
                n0 = cnt_ref[0]
                plsc.store_compressed(segl.at[pl.ds(n0, L)], seg, mask=inb)
                plsc.store_compressed(pidxl.at[pl.ds(n0, L)], pidx, mask=inb)
                cnt_ref[0] = n0 + nk

            @pl.when(cnt_ref[0] > FLUSH_AT)
            def _flush_now():
              flush(cnt_ref[0])
          return _
        lax.fori_loop(0, BLK // (L * SUB), chunk_body, None)

      blk_copy(0, 0, sem0).start()

      def pair_body(bb, _):
        b0 = 2 * bb
        blk_copy(b0 + 1, 1, sem1).start()
        blk_copy(b0, 0, sem0).wait()
        scan_block(0, b0 * BLK)

        @pl.when(b0 + 2 < nblk)
        def _prefetch():
          blk_copy(b0 + 2, 0, sem0).start()
        blk_copy(b0 + 1, 1, sem1).wait()
        scan_block(1, (b0 + 1) * BLK)
        return _
      lax.fori_loop(0, nblk // 2, pair_body, None)

      @pl.when(cnt_ref[0] > 0)
      def _final_flush():
        flush(cnt_ref[0])

      def fin_body(i, _):
        for u in range(UNR):
          sl = pl.ds((i * UNR + u) * L, L)
          v = acc[sl]
          acc[sl] = jnp.where(v == NEG_INF, 0.0, v)
        return _
      lax.fori_loop(0, n_vec // UNR, fin_body, None)

      pltpu.sync_copy(acc, out_hbm.at[r])


def kernel(rois, pts, pts_feature):
  nrois = rois.shape[0]
  npoints = pts.shape[0]
  C = pts_feature.shape[1]
  assert npoints % (2 * BLK) == 0 and C % L == 0
  n_rounds = -(-nrois // NW)
  nblk = npoints // BLK

  cx, cy, cz = rois[:, 0], rois[:, 1], rois[:, 2]
  dx, dy, dz = rois[:, 3], rois[:, 4], rois[:, 5]
  rz = rois[:, 6]
  czc = cz + dz * 0.5
  cosa = jnp.cos(-rz)
  sina = jnp.sin(-rz)
  hdx, hdy, hdz = dx * 0.5, dy * 0.5, dz * 0.5
  ixres = OUT_SIZE / dx
  iyres = OUT_SIZE / dy
  izres = OUT_SIZE / dz
  pad = jnp.zeros((nrois,), jnp.float32)
  prm = jnp.stack(
      [cx, cy, czc, hdx, hdy, hdz, cosa, sina, ixres, iyres, izres, cz,
       pad, pad, pad, pad], axis=1)

  coords = jnp.transpose(pts.T.reshape(3, nblk, BLK), (1, 0, 2))

  mesh = plsc.VectorSubcoreMesh(
      core_axis_name="c", subcore_axis_name="s",
      num_cores=NC, num_subcores=NS)

  fn = pl.kernel(
      functools.partial(_pool_body, nrois, npoints, C, n_rounds),
      out_type=jax.ShapeDtypeStruct((nrois, V * C), jnp.float32),
      mesh=mesh,
      compiler_params=pltpu.CompilerParams(
          needs_layout_passes=False, use_tc_tiling_on_sc=False),
      scratch_types=[
          pltpu.VMEM((V * C,), jnp.float32),
          pltpu.VMEM((2, 3, BLK), jnp.float32),
          pltpu.VMEM((CAP, C), jnp.float32),
          pltpu.VMEM((L,), jnp.float32),
          pltpu.VMEM((CAP,), jnp.int32),
          pltpu.VMEM((CAP,), jnp.int32),
          pltpu.SMEM((1,), jnp.int32),
          pltpu.SemaphoreType.DMA,
          pltpu.SemaphoreType.DMA,
          pltpu.SemaphoreType.DMA,
      ],
  )
  out = fn(prm, coords, pts_feature)
  return out.reshape(nrois, OUT_SIZE, OUT_SIZE, OUT_SIZE, C)
